# baseline (speedup 1.0000x reference)
import functools

import jax
import jax.numpy as jnp
from jax import lax
from jax.experimental import pallas as pl
from jax.experimental.pallas import tpu as pltpu
from jax.experimental.pallas import tpu_sc as plsc

_EMBED = 64
_Q = 4
_QROW = _Q * _EMBED
_CHUNK = 128
_NBUF = 2


def _sc_lookup(qidx, combo):
    nchunks_total = qidx.shape[0]
    nq = nchunks_total * _CHUNK
    info = plsc.get_sparse_core_info()
    ncores, nsub = info.num_cores, info.num_subcores
    nw = ncores * nsub
    g_per_w = nchunks_total // nw
    n_outer = g_per_w // _NBUF
    mesh = plsc.VectorSubcoreMesh(core_axis_name="c", subcore_axis_name="s")

    @functools.partial(
        pl.kernel,
        mesh=mesh,
        out_type=jax.ShapeDtypeStruct((nq, _QROW), jnp.float32),
        scratch_types=[
            pltpu.VMEM((_NBUF, _CHUNK), jnp.int32),
            pltpu.VMEM((_NBUF, _CHUNK, _QROW), jnp.float32),
            pltpu.SemaphoreType.DMA,
            pltpu.SemaphoreType.DMA,
            pltpu.SemaphoreType.DMA,
            pltpu.SemaphoreType.DMA,
            pltpu.SemaphoreType.DMA,
            pltpu.SemaphoreType.DMA,
        ],
    )
    def k(qidx_hbm, combo_hbm, out_hbm, qidx_v, rows_v,
          si0, si1, sg0, sg1, so0, so1):
        sem_i, sem_g, sem_o = (si0, si1), (sg0, sg1), (so0, so1)
        wid = lax.axis_index("s") * ncores + lax.axis_index("c")
        w_chunk0 = wid * g_per_w

        def fire_idx(g, b):
            pltpu.async_copy(qidx_hbm.at[w_chunk0 + g], qidx_v.at[b],
                             sem_i[b])

        fire_idx(0, 0)
        fire_idx(1, 1)

        def body(it, carry):
            for b in range(_NBUF):
                g = it * _NBUF + b
                pltpu.make_async_copy(qidx_hbm.at[w_chunk0 + g],
                                      qidx_v.at[b], sem_i[b]).wait()

                @pl.when(it < n_outer - 1)
                def _prefetch():
                    fire_idx(it * _NBUF + b + _NBUF, b)

                @pl.when(it >= 1)
                def _drain_out():
                    pltpu.make_async_copy(out_hbm.at[pl.ds(0, _CHUNK)],
                                          rows_v.at[b], sem_o[b]).wait()

                pltpu.async_copy(combo_hbm.at[qidx_v.at[b]], rows_v.at[b],
                                 sem_g[b]).wait()
                base = (w_chunk0 + g) * _CHUNK
                pltpu.async_copy(rows_v.at[b],
                                 out_hbm.at[pl.ds(base, _CHUNK)], sem_o[b])
            return carry

        lax.fori_loop(0, n_outer, body, 0)
        for b in range(_NBUF):
            pltpu.make_async_copy(out_hbm.at[pl.ds(0, _CHUNK)],
                                  rows_v.at[b], sem_o[b]).wait()

    return k(qidx, combo)


def _quad_table(table):
    q = jnp.arange(81)
    rows = [table[(q // (3 ** (3 - k))) % 3] for k in range(_Q)]
    return jnp.concatenate(rows, axis=1)


def kernel(speakers, table):
    b, h = speakers.shape
    nq = (b * h) // _Q
    s = speakers.reshape(nq, _Q).astype(jnp.int32)
    qidx = ((s[:, 0] * 3 + s[:, 1]) * 3 + s[:, 2]) * 3 + s[:, 3]
    combo = _quad_table(table)
    out = _sc_lookup(qidx.reshape(nq // _CHUNK, _CHUNK), combo)
    return out.reshape(b, h, _EMBED)
